# Initial kernel scaffold; baseline (speedup 1.0000x reference)
import jax
import jax.numpy as jnp
from jax.experimental import pallas as pl


def kernel(x, word_table, pos_table):
    raise NotImplementedError("write your pallas kernel here")


# speedup vs baseline: 3.4539x; 3.4539x over previous
import functools
import jax
import jax.numpy as jnp
from jax import lax
from jax.experimental import pallas as pl
from jax.experimental.pallas import tpu as pltpu
from jax.experimental.pallas import tpu_sc as plsc

NC = 2
NS = 16
NW = NC * NS
L = 16

VOCAB = 100000
D = 64
SEQ = 200
BATCH = 4096

IDX_W = 100
KROWS = 4
CHUNK = KROWS * IDX_W
N_FLAT = BATCH * SEQ
ROWS_TOTAL = N_FLAT // IDX_W
CHUNKS_PER_W = ROWS_TOTAL // (KROWS * NW)


def _body(x_hbm, word_hbm, pos_hbm, out_hbm, pos_v, idx_v, rows_v, sem):
    wid = lax.axis_index("s") * NC + lax.axis_index("c")

    pltpu.sync_copy(pos_hbm, pos_v)

    def chunk_body(g, _):
        r0 = (wid * CHUNKS_PER_W + g) * KROWS
        pltpu.sync_copy(x_hbm.at[pl.ds(r0, KROWS)], idx_v)
        cps = []
        for j in range(KROWS):
            cps.append(
                pltpu.async_copy(
                    word_hbm.at[idx_v.at[j]],
                    rows_v.at[pl.ds(j * IDX_W, IDX_W)],
                    sem,
                )
            )
        for cp in cps:
            cp.wait()

        def add_body(s, _):
            for d in range(D // L):
                p = pos_v[s, pl.ds(d * L, L)]
                for j in range(CHUNK // SEQ):
                    r = j * SEQ + s
                    rows_v[r, pl.ds(d * L, L)] = rows_v[r, pl.ds(d * L, L)] + p
            return ()

        lax.fori_loop(0, SEQ, add_body, ())

        pltpu.sync_copy(rows_v, out_hbm.at[pl.ds(r0 * IDX_W, CHUNK)])
        return ()

    lax.fori_loop(0, CHUNKS_PER_W, chunk_body, ())


@jax.jit
def kernel(x, word_table, pos_table):
    x_flat = x.reshape(ROWS_TOTAL, IDX_W)
    mesh = plsc.VectorSubcoreMesh(core_axis_name="c", subcore_axis_name="s")
    out = pl.kernel(
        _body,
        out_type=jax.ShapeDtypeStruct((N_FLAT, D), jnp.float32),
        mesh=mesh,
        compiler_params=pltpu.CompilerParams(use_tc_tiling_on_sc=False),
        scratch_types=[
            pltpu.VMEM((SEQ, D), jnp.float32),
            pltpu.VMEM((KROWS, IDX_W), jnp.int32),
            pltpu.VMEM((CHUNK, D), jnp.float32),
            pltpu.SemaphoreType.DMA,
        ],
    )(x_flat, word_table, pos_table)
    return out.reshape(BATCH, SEQ, D)
